# speedup vs baseline: 75.2377x; 75.2377x over previous
import functools

import jax
import jax.numpy as jnp
from jax import lax
from jax.experimental import pallas as pl
from jax.experimental.pallas import tpu as pltpu
from jax.experimental.pallas import tpu_sc as plsc

_B = 2
_C = 96
_H = 224
_W = 224
_L = _H * _W
_NP = _B * _C
_NW = 32
_PPW = _NP // _NW
_R = 16
_NCH = _H // _R
_CHW = _R * _W
_GRD = 8
_SROW = _GRD + _CHW + _GRD
_GPR = _W // 16

_mesh = plsc.VectorSubcoreMesh(core_axis_name="c", subcore_axis_name="s")


@functools.partial(
    pl.kernel,
    mesh=_mesh,
    out_type=jax.ShapeDtypeStruct((_NP * _L,), jnp.float32),
    scratch_types=[
        pltpu.VMEM((9 * _SROW,), jnp.float32),
        pltpu.VMEM((_CHW,), jnp.float32),
        pltpu.SemaphoreType.DMA,
    ],
)
def _fold(x_hbm, out_hbm, stage, outbuf, sem):
    cid = lax.axis_index("c")
    sid = lax.axis_index("s")
    wid = sid * 2 + cid

    zero16 = jnp.zeros((16,), jnp.float32)
    lane = lax.iota(jnp.int32, 16)
    m_lane0 = lane == 0
    m_lane15 = lane == 15

    def plane_body(pi, carry):
        plane = wid * _PPW + pi
        pbase = plane * (9 * _L)

        def chunk_body(ci, carry):
            h0 = ci * _R
            copies = []
            for s in range(9):
                kh = s // 3
                src = pbase + s * _L + (h0 + 1 - kh) * _W
                copies.append(
                    pltpu.async_copy(
                        x_hbm.at[pl.ds(src, _CHW)],
                        stage.at[pl.ds(s * _SROW + _GRD, _CHW)],
                        sem,
                    )
                )
            for cp in copies:
                cp.wait()

            @pl.when(ci == 0)
            def _():
                for s in (6, 7, 8):
                    for g in range(_GPR):
                        stage[pl.ds(s * _SROW + _GRD + 16 * g, 16)] = zero16

            @pl.when(ci == _NCH - 1)
            def _():
                for s in (0, 1, 2):
                    base = _GRD + (_R - 1) * _W
                    for g in range(_GPR):
                        stage[pl.ds(s * _SROW + base + 16 * g, 16)] = zero16

            def row_body(r, carry):
                off = _GRD + r * _W
                for g in range(_GPR):
                    o = off + 16 * g
                    acc = stage[pl.ds(1 * _SROW + o, 16)]
                    acc = acc + stage[pl.ds(4 * _SROW + o, 16)]
                    acc = acc + stage[pl.ds(7 * _SROW + o, 16)]
                    for s in (0, 3, 6):
                        v = stage[pl.ds(s * _SROW + o + 1, 16)]
                        if g == _GPR - 1:
                            v = jnp.where(m_lane15, 0.0, v)
                        acc = acc + v
                    for s in (2, 5, 8):
                        v = stage[pl.ds(s * _SROW + o - 1, 16)]
                        if g == 0:
                            v = jnp.where(m_lane0, 0.0, v)
                        acc = acc + v
                    outbuf[pl.ds(r * _W + 16 * g, 16)] = acc
                return carry

            lax.fori_loop(0, _R, row_body, None)
            pltpu.sync_copy(outbuf, out_hbm.at[pl.ds(plane * _L + h0 * _W, _CHW)])
            return carry

        lax.fori_loop(0, _NCH, chunk_body, None)
        return carry

    lax.fori_loop(0, _PPW, plane_body, None)


def kernel(input):
    x = input.reshape(-1)
    out = _fold(x)
    return out.reshape(_B, _C, _H, _W)
